# baseline (speedup 1.0000x reference)
import functools

import jax
import jax.numpy as jnp
from jax import lax
from jax.experimental import pallas as pl
from jax.experimental.pallas import tpu as pltpu
from jax.experimental.pallas import tpu_sc as plsc

N = 10000
K = 32
D = 128
OUT = 128

NW = 32
B = 4
NBT = N // B
ROWS = B * K
TMAX = (NBT + NW - 1) // NW
TFULL = NBT // NW
LANES = 16


def _gather_max(x, ei_flat):
    mesh = plsc.VectorSubcoreMesh(core_axis_name="c", subcore_axis_name="s")

    @functools.partial(
        pl.kernel,
        mesh=mesh,
        out_type=jax.ShapeDtypeStruct((N, D), jnp.float32),
        scratch_types=[
            pltpu.VMEM_SHARED((N, D), jnp.float32),
            pltpu.VMEM((TMAX * ROWS,), jnp.int32),
            pltpu.VMEM((ROWS, D), jnp.float32),
            pltpu.VMEM((ROWS, D), jnp.float32),
            pltpu.VMEM((B, D), jnp.float32),
            pltpu.VMEM((B, D), jnp.float32),
            pltpu.SemaphoreType.DMA,
            pltpu.SemaphoreType.DMA,
            pltpu.SemaphoreType.DMA,
            pltpu.SemaphoreType.DMA,
            pltpu.SemaphoreType.DMA,
        ],
    )
    def k(x_hbm, ei_hbm, out_hbm, x_sh, idx_v, gb0, gb1, ms0, ms1,
          isem, gs0, gs1, os0, os1):
        sid = lax.axis_index("s")
        w = sid * 2 + lax.axis_index("c")
        gbuf = (gb0, gb1)
        gsem = (gs0, gs1)
        mst = (ms0, ms1)
        osem = (os0, os1)

        for t in range(TFULL):
            pltpu.async_copy(
                ei_hbm.at[pl.ds((w + NW * t) * ROWS, ROWS)],
                idx_v.at[pl.ds(t * ROWS, ROWS)],
                isem,
            )

        @pl.when(w + NW * TFULL < NBT)
        def _tail_idx():
            pltpu.async_copy(
                ei_hbm.at[pl.ds((w + NW * TFULL) * ROWS, ROWS)],
                idx_v.at[pl.ds(TFULL * ROWS, ROWS)],
                isem,
            )

        pltpu.sync_copy(
            x_hbm.at[pl.ds(sid * 624, 624)],
            x_sh.at[pl.ds(sid * 624, 624)],
        )

        @pl.when(sid == 0)
        def _stage_tail():
            pltpu.sync_copy(
                x_hbm.at[pl.ds(16 * 624, N - 16 * 624)],
                x_sh.at[pl.ds(16 * 624, N - 16 * 624)],
            )

        pltpu.make_async_copy(
            ei_hbm.at[pl.ds(0, TFULL * ROWS)],
            idx_v.at[pl.ds(0, TFULL * ROWS)],
            isem,
        ).wait()

        @pl.when(w + NW * TFULL < NBT)
        def _tail_idx_wait():
            pltpu.make_async_copy(
                ei_hbm.at[pl.ds(0, ROWS)],
                idx_v.at[pl.ds(TFULL * ROWS, ROWS)],
                isem,
            ).wait()

        plsc.subcore_barrier()

        def fire(t, p):
            pltpu.async_copy(
                x_sh.at[idx_v.at[pl.ds(t * ROWS, ROWS)]], gbuf[p], gsem[p]
            )

        fire(0, 0)
        fire(1, 1)

        def slot(t, p):
            active = w + NW * t < NBT

            @pl.when(active)
            def _():
                pltpu.make_async_copy(
                    x_sh.at[idx_v.at[pl.ds(0, ROWS)]], gbuf[p], gsem[p]
                ).wait()

                @pl.when(t >= 2)
                def _reclaim():
                    pltpu.make_async_copy(
                        mst[p], out_hbm.at[pl.ds(0, B)], osem[p]
                    ).wait()

                def node_body(j):
                    buf = gbuf[p]
                    for v in range(D // LANES):
                        sl = pl.ds(v * LANES, LANES)
                        acc = buf[j * K, sl]
                        for kk in range(1, K):
                            acc = jnp.maximum(acc, buf[j * K + kk, sl])
                        mst[p][j, sl] = acc

                pl.loop(0, B)(node_body)

                pltpu.async_copy(
                    mst[p], out_hbm.at[pl.ds((w + NW * t) * B, B)], osem[p]
                )

                @pl.when(w + NW * (t + 2) < NBT)
                def _fire_next():
                    fire(t + 2, p)

        def pair(tp):
            slot(tp * 2, 0)
            slot(tp * 2 + 1, 1)

        pl.loop(0, (TMAX + 1) // 2)(pair)

        for p in range(2):
            pltpu.make_async_copy(
                mst[p], out_hbm.at[pl.ds(0, B)], osem[p]
            ).wait()

    return k(x, ei_flat)


R_BLK = 1000


def _mlp_pre(x, W, b):
    grid = (N // R_BLK,)

    def body(x_ref, w_ref, b_ref, o_ref):
        wd = w_ref[:D, :] - w_ref[D:, :]
        acc = jnp.dot(x_ref[...], wd, preferred_element_type=jnp.float32,
                      precision=lax.Precision.HIGHEST)
        o_ref[...] = acc + b_ref[...]

    return pl.pallas_call(
        body,
        grid=grid,
        in_specs=[
            pl.BlockSpec((R_BLK, D), lambda i: (i, 0)),
            pl.BlockSpec((2 * D, OUT), lambda i: (0, 0)),
            pl.BlockSpec((1, OUT), lambda i: (0, 0)),
        ],
        out_specs=pl.BlockSpec((R_BLK, OUT), lambda i: (i, 0)),
        out_shape=jax.ShapeDtypeStruct((N, OUT), jnp.float32),
    )(x, W, b.reshape(1, OUT))


def _mlp_post(e, m, W):
    grid = (N // R_BLK,)

    def body(e_ref, m_ref, w_ref, o_ref):
        acc = jnp.dot(m_ref[...], w_ref[D:, :],
                      preferred_element_type=jnp.float32,
                      precision=lax.Precision.HIGHEST)
        o_ref[...] = acc + e_ref[...]

    return pl.pallas_call(
        body,
        grid=grid,
        in_specs=[
            pl.BlockSpec((R_BLK, OUT), lambda i: (i, 0)),
            pl.BlockSpec((R_BLK, D), lambda i: (i, 0)),
            pl.BlockSpec((2 * D, OUT), lambda i: (0, 0)),
        ],
        out_specs=pl.BlockSpec((R_BLK, OUT), lambda i: (i, 0)),
        out_shape=jax.ShapeDtypeStruct((N, OUT), jnp.float32),
    )(e, m, W)


def kernel(x, edge_index, W, b):
    ei_flat = edge_index.astype(jnp.int32).reshape(-1)
    m = _gather_max(x, ei_flat)
    e = _mlp_pre(x, W, b)
    return _mlp_post(e, m, W)


# speedup vs baseline: 1.0878x; 1.0878x over previous
import functools

import jax
import jax.numpy as jnp
from jax import lax
from jax.experimental import pallas as pl
from jax.experimental.pallas import tpu as pltpu
from jax.experimental.pallas import tpu_sc as plsc

N = 10000
K = 32
D = 128
OUT = 128

NW = 32
B = 4
NBT = N // B
ROWS = B * K
TMAX = (NBT + NW - 1) // NW
TFULL = NBT // NW
LANES = 16


def _gather_max(x, ei_flat):
    mesh = plsc.VectorSubcoreMesh(core_axis_name="c", subcore_axis_name="s")

    @functools.partial(
        pl.kernel,
        mesh=mesh,
        out_type=jax.ShapeDtypeStruct((N, D), jnp.float32),
        scratch_types=[
            pltpu.VMEM_SHARED((N, D), jnp.float32),
            pltpu.VMEM((TMAX * ROWS,), jnp.int32),
            pltpu.VMEM((ROWS, D), jnp.float32),
            pltpu.VMEM((ROWS, D), jnp.float32),
            pltpu.VMEM((B, D), jnp.float32),
            pltpu.VMEM((B, D), jnp.float32),
            pltpu.SemaphoreType.DMA,
            pltpu.SemaphoreType.DMA,
            pltpu.SemaphoreType.DMA,
            pltpu.SemaphoreType.DMA,
            pltpu.SemaphoreType.DMA,
        ],
    )
    def k(x_hbm, ei_hbm, out_hbm, x_sh, idx_v, gb0, gb1, ms0, ms1,
          isem, gs0, gs1, os0, os1):
        sid = lax.axis_index("s")
        w = sid * 2 + lax.axis_index("c")
        gbuf = (gb0, gb1)
        gsem = (gs0, gs1)
        mst = (ms0, ms1)
        osem = (os0, os1)

        for t in range(TFULL):
            pltpu.async_copy(
                ei_hbm.at[pl.ds((w + NW * t) * ROWS, ROWS)],
                idx_v.at[pl.ds(t * ROWS, ROWS)],
                isem,
            )

        @pl.when(w + NW * TFULL < NBT)
        def _tail_idx():
            pltpu.async_copy(
                ei_hbm.at[pl.ds((w + NW * TFULL) * ROWS, ROWS)],
                idx_v.at[pl.ds(TFULL * ROWS, ROWS)],
                isem,
            )

        pltpu.sync_copy(
            x_hbm.at[pl.ds(sid * 624, 624)],
            x_sh.at[pl.ds(sid * 624, 624)],
        )

        @pl.when(sid == 0)
        def _stage_tail():
            pltpu.sync_copy(
                x_hbm.at[pl.ds(16 * 624, N - 16 * 624)],
                x_sh.at[pl.ds(16 * 624, N - 16 * 624)],
            )

        pltpu.make_async_copy(
            ei_hbm.at[pl.ds(0, TFULL * ROWS)],
            idx_v.at[pl.ds(0, TFULL * ROWS)],
            isem,
        ).wait()

        @pl.when(w + NW * TFULL < NBT)
        def _tail_idx_wait():
            pltpu.make_async_copy(
                ei_hbm.at[pl.ds(0, ROWS)],
                idx_v.at[pl.ds(TFULL * ROWS, ROWS)],
                isem,
            ).wait()

        plsc.subcore_barrier()

        def fire(t, p):
            from_hbm = lax.rem(t, 6) == 5

            @pl.when(from_hbm)
            def _():
                pltpu.async_copy(
                    x_hbm.at[idx_v.at[pl.ds(t * ROWS, ROWS)]],
                    gbuf[p], gsem[p],
                )

            @pl.when(jnp.logical_not(from_hbm))
            def _():
                pltpu.async_copy(
                    x_sh.at[idx_v.at[pl.ds(t * ROWS, ROWS)]],
                    gbuf[p], gsem[p],
                )

        fire(0, 0)
        fire(1, 1)

        def slot(t, p):
            active = w + NW * t < NBT

            @pl.when(active)
            def _():
                pltpu.make_async_copy(
                    x_sh.at[idx_v.at[pl.ds(0, ROWS)]], gbuf[p], gsem[p]
                ).wait()

                @pl.when(t >= 2)
                def _reclaim():
                    pltpu.make_async_copy(
                        mst[p], out_hbm.at[pl.ds(0, B)], osem[p]
                    ).wait()

                def node_body(j):
                    buf = gbuf[p]
                    for v in range(D // LANES):
                        sl = pl.ds(v * LANES, LANES)
                        acc = buf[j * K, sl]
                        for kk in range(1, K):
                            acc = jnp.maximum(acc, buf[j * K + kk, sl])
                        mst[p][j, sl] = acc

                pl.loop(0, B)(node_body)

                pltpu.async_copy(
                    mst[p], out_hbm.at[pl.ds((w + NW * t) * B, B)], osem[p]
                )

                @pl.when(w + NW * (t + 2) < NBT)
                def _fire_next():
                    fire(t + 2, p)

        def pair(tp):
            slot(tp * 2, 0)
            slot(tp * 2 + 1, 1)

        pl.loop(0, (TMAX + 1) // 2)(pair)

        for p in range(2):
            pltpu.make_async_copy(
                mst[p], out_hbm.at[pl.ds(0, B)], osem[p]
            ).wait()

    return k(x, ei_flat)


def _mlp(x, m, W, b):
    R = 1000
    grid = (N // R,)

    def body(x_ref, m_ref, w_ref, b_ref, o_ref):
        xb = x_ref[...]
        mb = m_ref[...]
        w1 = w_ref[:D, :]
        w2 = w_ref[D:, :]
        acc = jnp.dot(xb, w1, preferred_element_type=jnp.float32)
        acc += jnp.dot(mb - xb, w2, preferred_element_type=jnp.float32)
        o_ref[...] = acc + b_ref[...]

    return pl.pallas_call(
        body,
        grid=grid,
        in_specs=[
            pl.BlockSpec((R, D), lambda i: (i, 0)),
            pl.BlockSpec((R, D), lambda i: (i, 0)),
            pl.BlockSpec((2 * D, OUT), lambda i: (0, 0)),
            pl.BlockSpec((1, OUT), lambda i: (0, 0)),
        ],
        out_specs=pl.BlockSpec((R, OUT), lambda i: (i, 0)),
        out_shape=jax.ShapeDtypeStruct((N, OUT), jnp.float32),
    )(x, m, W, b.reshape(1, OUT))


def kernel(x, edge_index, W, b):
    ei_flat = edge_index.astype(jnp.int32).reshape(-1)
    m = _gather_max(x, ei_flat)
    return _mlp(x, m, W, b)


# speedup vs baseline: 1.1228x; 1.0322x over previous
import functools

import jax
import jax.numpy as jnp
from jax import lax
from jax.experimental import pallas as pl
from jax.experimental.pallas import tpu as pltpu
from jax.experimental.pallas import tpu_sc as plsc

N = 10000
K = 32
D = 128
OUT = 128

NW = 32
B = 4
NBT = N // B
ROWS = B * K
TMAX = (NBT + NW - 1) // NW
TFULL = NBT // NW
LANES = 16


def _gather_max(x, ei_flat):
    mesh = plsc.VectorSubcoreMesh(core_axis_name="c", subcore_axis_name="s")

    @functools.partial(
        pl.kernel,
        mesh=mesh,
        out_type=jax.ShapeDtypeStruct((N, D), jnp.float32),
        scratch_types=[
            pltpu.VMEM_SHARED((N, D), jnp.float32),
            pltpu.VMEM((TMAX * ROWS,), jnp.int32),
            pltpu.VMEM((ROWS, D), jnp.float32),
            pltpu.VMEM((ROWS, D), jnp.float32),
            pltpu.VMEM((B, D), jnp.float32),
            pltpu.VMEM((B, D), jnp.float32),
            pltpu.SemaphoreType.DMA,
            pltpu.SemaphoreType.DMA,
            pltpu.SemaphoreType.DMA,
            pltpu.SemaphoreType.DMA,
            pltpu.SemaphoreType.DMA,
        ],
    )
    def k(x_hbm, ei_hbm, out_hbm, x_sh, idx_v, gb0, gb1, ms0, ms1,
          isem, gs0, gs1, os0, os1):
        sid = lax.axis_index("s")
        w = sid * 2 + lax.axis_index("c")
        gbuf = (gb0, gb1)
        gsem = (gs0, gs1)
        mst = (ms0, ms1)
        osem = (os0, os1)

        for t in range(TFULL):
            pltpu.async_copy(
                ei_hbm.at[pl.ds((w + NW * t) * ROWS, ROWS)],
                idx_v.at[pl.ds(t * ROWS, ROWS)],
                isem,
            )

        @pl.when(w + NW * TFULL < NBT)
        def _tail_idx():
            pltpu.async_copy(
                ei_hbm.at[pl.ds((w + NW * TFULL) * ROWS, ROWS)],
                idx_v.at[pl.ds(TFULL * ROWS, ROWS)],
                isem,
            )

        pltpu.sync_copy(
            x_hbm.at[pl.ds(sid * 624, 624)],
            x_sh.at[pl.ds(sid * 624, 624)],
        )

        @pl.when(sid == 0)
        def _stage_tail():
            pltpu.sync_copy(
                x_hbm.at[pl.ds(16 * 624, N - 16 * 624)],
                x_sh.at[pl.ds(16 * 624, N - 16 * 624)],
            )

        pltpu.make_async_copy(
            ei_hbm.at[pl.ds(0, TFULL * ROWS)],
            idx_v.at[pl.ds(0, TFULL * ROWS)],
            isem,
        ).wait()

        @pl.when(w + NW * TFULL < NBT)
        def _tail_idx_wait():
            pltpu.make_async_copy(
                ei_hbm.at[pl.ds(0, ROWS)],
                idx_v.at[pl.ds(TFULL * ROWS, ROWS)],
                isem,
            ).wait()

        plsc.subcore_barrier()

        def fire(t, p):
            from_hbm = lax.rem(t, 4) == 3

            @pl.when(from_hbm)
            def _():
                pltpu.async_copy(
                    x_hbm.at[idx_v.at[pl.ds(t * ROWS, ROWS)]],
                    gbuf[p], gsem[p],
                )

            @pl.when(jnp.logical_not(from_hbm))
            def _():
                pltpu.async_copy(
                    x_sh.at[idx_v.at[pl.ds(t * ROWS, ROWS)]],
                    gbuf[p], gsem[p],
                )

        fire(0, 0)
        fire(1, 1)

        def slot(t, p):
            active = w + NW * t < NBT

            @pl.when(active)
            def _():
                pltpu.make_async_copy(
                    x_sh.at[idx_v.at[pl.ds(0, ROWS)]], gbuf[p], gsem[p]
                ).wait()

                @pl.when(t >= 2)
                def _reclaim():
                    pltpu.make_async_copy(
                        mst[p], out_hbm.at[pl.ds(0, B)], osem[p]
                    ).wait()

                def node_body(j):
                    buf = gbuf[p]
                    for v in range(D // LANES):
                        sl = pl.ds(v * LANES, LANES)
                        acc = buf[j * K, sl]
                        for kk in range(1, K):
                            acc = jnp.maximum(acc, buf[j * K + kk, sl])
                        mst[p][j, sl] = acc

                pl.loop(0, B)(node_body)

                pltpu.async_copy(
                    mst[p], out_hbm.at[pl.ds((w + NW * t) * B, B)], osem[p]
                )

                @pl.when(w + NW * (t + 2) < NBT)
                def _fire_next():
                    fire(t + 2, p)

        def pair(tp):
            slot(tp * 2, 0)
            slot(tp * 2 + 1, 1)

        pl.loop(0, (TMAX + 1) // 2)(pair)

        for p in range(2):
            pltpu.make_async_copy(
                mst[p], out_hbm.at[pl.ds(0, B)], osem[p]
            ).wait()

    return k(x, ei_flat)


def _mlp(x, m, W, b):
    R = 1000
    grid = (N // R,)

    def body(x_ref, m_ref, w_ref, b_ref, o_ref):
        xb = x_ref[...]
        mb = m_ref[...]
        w1 = w_ref[:D, :]
        w2 = w_ref[D:, :]
        acc = jnp.dot(xb, w1, preferred_element_type=jnp.float32)
        acc += jnp.dot(mb - xb, w2, preferred_element_type=jnp.float32)
        o_ref[...] = acc + b_ref[...]

    return pl.pallas_call(
        body,
        grid=grid,
        in_specs=[
            pl.BlockSpec((R, D), lambda i: (i, 0)),
            pl.BlockSpec((R, D), lambda i: (i, 0)),
            pl.BlockSpec((2 * D, OUT), lambda i: (0, 0)),
            pl.BlockSpec((1, OUT), lambda i: (0, 0)),
        ],
        out_specs=pl.BlockSpec((R, OUT), lambda i: (i, 0)),
        out_shape=jax.ShapeDtypeStruct((N, OUT), jnp.float32),
    )(x, m, W, b.reshape(1, OUT))


def kernel(x, edge_index, W, b):
    ei_flat = edge_index.astype(jnp.int32).reshape(-1)
    m = _gather_max(x, ei_flat)
    return _mlp(x, m, W, b)


# speedup vs baseline: 1.1594x; 1.0327x over previous
import functools

import jax
import jax.numpy as jnp
from jax import lax
from jax.experimental import pallas as pl
from jax.experimental.pallas import tpu as pltpu
from jax.experimental.pallas import tpu_sc as plsc

N = 10000
K = 32
D = 128
OUT = 128

NW = 32
B = 4
NBT = N // B
ROWS = B * K
TMAX = (NBT + NW - 1) // NW
TFULL = NBT // NW
LANES = 16


def _gather_max(x, ei_flat):
    mesh = plsc.VectorSubcoreMesh(core_axis_name="c", subcore_axis_name="s")

    @functools.partial(
        pl.kernel,
        mesh=mesh,
        out_type=jax.ShapeDtypeStruct((N, D), jnp.float32),
        scratch_types=[
            pltpu.VMEM_SHARED((N, D), jnp.float32),
            pltpu.VMEM((TMAX * ROWS,), jnp.int32),
            pltpu.VMEM((ROWS, D), jnp.float32),
            pltpu.VMEM((ROWS, D), jnp.float32),
            pltpu.VMEM((B, D), jnp.float32),
            pltpu.VMEM((B, D), jnp.float32),
            pltpu.SemaphoreType.DMA,
            pltpu.SemaphoreType.DMA,
            pltpu.SemaphoreType.DMA,
            pltpu.SemaphoreType.DMA,
            pltpu.SemaphoreType.DMA,
        ],
    )
    def k(x_hbm, ei_hbm, out_hbm, x_sh, idx_v, gb0, gb1, ms0, ms1,
          isem, gs0, gs1, os0, os1):
        sid = lax.axis_index("s")
        w = sid * 2 + lax.axis_index("c")
        gbuf = (gb0, gb1)
        gsem = (gs0, gs1)
        mst = (ms0, ms1)
        osem = (os0, os1)

        for t in range(TFULL):
            pltpu.async_copy(
                ei_hbm.at[pl.ds((w + NW * t) * ROWS, ROWS)],
                idx_v.at[pl.ds(t * ROWS, ROWS)],
                isem,
            )

        @pl.when(w + NW * TFULL < NBT)
        def _tail_idx():
            pltpu.async_copy(
                ei_hbm.at[pl.ds((w + NW * TFULL) * ROWS, ROWS)],
                idx_v.at[pl.ds(TFULL * ROWS, ROWS)],
                isem,
            )

        pltpu.sync_copy(
            x_hbm.at[pl.ds(sid * 624, 624)],
            x_sh.at[pl.ds(sid * 624, 624)],
        )

        @pl.when(sid == 0)
        def _stage_tail():
            pltpu.sync_copy(
                x_hbm.at[pl.ds(16 * 624, N - 16 * 624)],
                x_sh.at[pl.ds(16 * 624, N - 16 * 624)],
            )

        pltpu.make_async_copy(
            ei_hbm.at[pl.ds(0, TFULL * ROWS)],
            idx_v.at[pl.ds(0, TFULL * ROWS)],
            isem,
        ).wait()

        @pl.when(w + NW * TFULL < NBT)
        def _tail_idx_wait():
            pltpu.make_async_copy(
                ei_hbm.at[pl.ds(0, ROWS)],
                idx_v.at[pl.ds(TFULL * ROWS, ROWS)],
                isem,
            ).wait()

        plsc.subcore_barrier()

        def fire(t, p):
            from_hbm = lax.rem(t, 3) == 2

            @pl.when(from_hbm)
            def _():
                pltpu.async_copy(
                    x_hbm.at[idx_v.at[pl.ds(t * ROWS, ROWS)]],
                    gbuf[p], gsem[p],
                )

            @pl.when(jnp.logical_not(from_hbm))
            def _():
                pltpu.async_copy(
                    x_sh.at[idx_v.at[pl.ds(t * ROWS, ROWS)]],
                    gbuf[p], gsem[p],
                )

        fire(0, 0)
        fire(1, 1)

        def slot(t, p):
            active = w + NW * t < NBT

            @pl.when(active)
            def _():
                pltpu.make_async_copy(
                    x_sh.at[idx_v.at[pl.ds(0, ROWS)]], gbuf[p], gsem[p]
                ).wait()

                @pl.when(t >= 2)
                def _reclaim():
                    pltpu.make_async_copy(
                        mst[p], out_hbm.at[pl.ds(0, B)], osem[p]
                    ).wait()

                def node_body(j):
                    buf = gbuf[p]
                    for v in range(D // LANES):
                        sl = pl.ds(v * LANES, LANES)
                        acc = buf[j * K, sl]
                        for kk in range(1, K):
                            acc = jnp.maximum(acc, buf[j * K + kk, sl])
                        mst[p][j, sl] = acc

                pl.loop(0, B)(node_body)

                pltpu.async_copy(
                    mst[p], out_hbm.at[pl.ds((w + NW * t) * B, B)], osem[p]
                )

                @pl.when(w + NW * (t + 2) < NBT)
                def _fire_next():
                    fire(t + 2, p)

        def pair(tp):
            slot(tp * 2, 0)
            slot(tp * 2 + 1, 1)

        pl.loop(0, (TMAX + 1) // 2)(pair)

        for p in range(2):
            pltpu.make_async_copy(
                mst[p], out_hbm.at[pl.ds(0, B)], osem[p]
            ).wait()

    return k(x, ei_flat)


def _mlp(x, m, W, b):
    R = 1000
    grid = (N // R,)

    def body(x_ref, m_ref, w_ref, b_ref, o_ref):
        xb = x_ref[...]
        mb = m_ref[...]
        w1 = w_ref[:D, :]
        w2 = w_ref[D:, :]
        acc = jnp.dot(xb, w1, preferred_element_type=jnp.float32)
        acc += jnp.dot(mb - xb, w2, preferred_element_type=jnp.float32)
        o_ref[...] = acc + b_ref[...]

    return pl.pallas_call(
        body,
        grid=grid,
        in_specs=[
            pl.BlockSpec((R, D), lambda i: (i, 0)),
            pl.BlockSpec((R, D), lambda i: (i, 0)),
            pl.BlockSpec((2 * D, OUT), lambda i: (0, 0)),
            pl.BlockSpec((1, OUT), lambda i: (0, 0)),
        ],
        out_specs=pl.BlockSpec((R, OUT), lambda i: (i, 0)),
        out_shape=jax.ShapeDtypeStruct((N, OUT), jnp.float32),
    )(x, m, W, b.reshape(1, OUT))


def kernel(x, edge_index, W, b):
    ei_flat = edge_index.astype(jnp.int32).reshape(-1)
    m = _gather_max(x, ei_flat)
    return _mlp(x, m, W, b)
